# baseline (speedup 1.0000x reference)
import jax
import jax.numpy as jnp
from jax import lax
from jax.experimental import pallas as pl
from jax.experimental.pallas import tpu as pltpu
from jax.experimental.pallas import tpu_sc as plsc

_VOCAB = 1000000
_DIM = 64
_BATCH = 16384
_HIST = 50
_HIST_PAD = 56
_TOTAL = _BATCH * _HIST
_NC = 2
_NS = 16
_NW = _NC * _NS
_PER_W = _TOTAL // _NW
_PER_WB = _BATCH // _NW
_CB = 16
_CHUNK = _CB * _HIST
_RAW = _CHUNK + 16
_N_CHUNKS = _PER_WB // _CB
_NBUF = 2


def _gather_body(table_hbm, idx_hbm, out_hbm,
                 raw0, raw1, idx0, idx1, rows0, rows1,
                 sem_i0, sem_i1, sem_g0, sem_g1, sem_s0, sem_s1):
    raw_v = (raw0, raw1)
    idx_v = (idx0, idx1)
    rows_v = (rows0, rows1)
    sem_i = (sem_i0, sem_i1)
    sem_g = (sem_g0, sem_g1)
    sem_s = (sem_s0, sem_s1)

    wid = lax.axis_index("s") * _NC + lax.axis_index("c")
    base = wid * _PER_W
    base_b = wid * _PER_WB
    iota = lax.iota(jnp.int32, 16)

    def load_raw(c, b):
        off = pl.multiple_of(base + c * _CHUNK, 8)
        pltpu.async_copy(idx_hbm.at[pl.ds(off, _RAW)], raw_v[b], sem_i[b])

    def wait_raw(c, b):
        off = pl.multiple_of(base + c * _CHUNK, 8)
        pltpu.make_async_copy(idx_hbm.at[pl.ds(off, _RAW)], raw_v[b],
                              sem_i[b]).wait()

    def restage(b):
        for j in range(_CB):
            for k in range(4):
                vals = plsc.load_gather(raw_v[b], [j * _HIST + k * 16 + iota])
                idx_v[b][pl.ds(j * _HIST_PAD + k * 16, 16)] = vals

    def store_rows(c, b):
        b0 = pl.multiple_of(base_b + c * _CB, 8)
        return pltpu.make_async_copy(
            rows_v[b].at[:, pl.ds(0, _HIST), :],
            out_hbm.at[pl.ds(b0, _CB)], sem_s[b])

    for b in range(_NBUF):
        load_raw(b, b)

    def step(g, carry):
        for b in range(_NBUF):
            c = g * _NBUF + b

            @pl.when(c >= _NBUF)
            def _():
                store_rows(c - _NBUF, b).wait()

            wait_raw(c, b)
            restage(b)

            for j in range(_CB):
                pltpu.async_copy(
                    table_hbm.at[idx_v[b].at[pl.ds(j * _HIST_PAD,
                                                   _HIST_PAD)]],
                    rows_v[b].at[j], sem_g[b])
            for j in range(_CB):
                pltpu.make_async_copy(
                    table_hbm.at[idx_v[b].at[pl.ds(j * _HIST_PAD,
                                                   _HIST_PAD)]],
                    rows_v[b].at[j], sem_g[b]).wait()

            @pl.when(c + _NBUF < _N_CHUNKS)
            def _():
                load_raw(c + _NBUF, b)

            store_rows(c, b).start()
        return carry

    lax.fori_loop(0, _N_CHUNKS // _NBUF, step, 0)

    for b in range(_NBUF):
        store_rows(_N_CHUNKS - _NBUF + b, b).wait()


def kernel(instrument_ids, embedding_table):
    idx_pad = jnp.pad(instrument_ids.reshape(_TOTAL), (0, 16))
    mesh = plsc.VectorSubcoreMesh(core_axis_name="c", subcore_axis_name="s")
    out = pl.kernel(
        _gather_body,
        out_type=jax.ShapeDtypeStruct((_BATCH, _HIST, _DIM), jnp.float32),
        mesh=mesh,
        scratch_types=[
            pltpu.VMEM((_RAW,), jnp.int32),
            pltpu.VMEM((_RAW,), jnp.int32),
            pltpu.VMEM((_CB * _HIST_PAD + 8,), jnp.int32),
            pltpu.VMEM((_CB * _HIST_PAD + 8,), jnp.int32),
            pltpu.VMEM((_CB, _HIST_PAD, _DIM), jnp.float32),
            pltpu.VMEM((_CB, _HIST_PAD, _DIM), jnp.float32),
            pltpu.SemaphoreType.DMA,
            pltpu.SemaphoreType.DMA,
            pltpu.SemaphoreType.DMA,
            pltpu.SemaphoreType.DMA,
            pltpu.SemaphoreType.DMA,
            pltpu.SemaphoreType.DMA,
        ],
        compiler_params=pltpu.CompilerParams(use_tc_tiling_on_sc=False,
                                             needs_layout_passes=False),
    )(embedding_table, idx_pad)
    return out
